# baseline (speedup 1.0000x reference)
import functools

import jax
import jax.numpy as jnp
from jax import lax
from jax.experimental import pallas as pl
from jax.experimental.pallas import tpu as pltpu
from jax.experimental.pallas import tpu_sc as plsc


def kernel(x, table):
    B, H = x.shape
    V, D = table.shape
    total = B * H

    info = plsc.get_sparse_core_info()
    NC = info.num_cores
    NW = NC * info.num_subcores
    b_per_w = total // NW
    CHUNK = 128
    n_chunks = b_per_w // CHUNK
    K = 2
    BLK = K * CHUNK
    NBUF = 4
    n_outer = n_chunks // K
    n_groups = n_outer // NBUF

    idx = x.reshape(NW, n_chunks, CHUNK)

    mesh = plsc.VectorSubcoreMesh(core_axis_name="c", subcore_axis_name="s")

    @functools.partial(
        pl.kernel,
        mesh=mesh,
        out_type=jax.ShapeDtypeStruct((total, D), table.dtype),
        compiler_params=pltpu.CompilerParams(use_tc_tiling_on_sc=False),
        scratch_types=[
            pltpu.VMEM((n_chunks, CHUNK), jnp.int32),
            pltpu.VMEM((NBUF * BLK, D), jnp.float32),
        ] + [pltpu.SemaphoreType.DMA] * (2 * NBUF),
    )
    def emb_kernel(idx_hbm, table_hbm, out_hbm, idx_v, rows_v, *sems):
        sem_g = sems[:NBUF]
        sem_w = sems[NBUF:]
        wid = lax.axis_index("s") * NC + lax.axis_index("c")
        pltpu.sync_copy(idx_hbm.at[wid], idx_v)
        base = wid * b_per_w

        def fire_gather(it, b):
            for j in range(K):
                pltpu.async_copy(
                    table_hbm.at[idx_v.at[it * K + j]],
                    rows_v.at[pl.ds(b * BLK + j * CHUNK, CHUNK)],
                    sem_g[b],
                )

        def wait_gather(b):
            for j in range(K):
                pltpu.make_async_copy(
                    table_hbm.at[idx_v.at[0]],
                    rows_v.at[pl.ds(b * BLK + j * CHUNK, CHUNK)],
                    sem_g[b],
                ).wait()

        def start_write(b, it):
            pltpu.async_copy(
                rows_v.at[pl.ds(b * BLK, BLK)],
                out_hbm.at[pl.ds(base + it * BLK, BLK)],
                sem_w[b],
            )

        def wait_write(b):
            pltpu.make_async_copy(
                rows_v.at[pl.ds(b * BLK, BLK)],
                out_hbm.at[pl.ds(0, BLK)],
                sem_w[b],
            ).wait()

        for b in range(NBUF):
            fire_gather(b, b)

        def body(q, carry):
            for b in range(NBUF):
                it = q * NBUF + b
                wait_gather(b)
                start_write(b, it)

                @pl.when(q < n_groups - 1)
                def _():
                    wait_write(b)
                    fire_gather(it + NBUF, b)

            return carry

        lax.fori_loop(0, n_groups, body, None)
        for b in range(NBUF):
            wait_write(b)

    out = emb_kernel(idx, table)
    return out.reshape(B, H, D)


# speedup vs baseline: 1.1780x; 1.1780x over previous
import functools

import jax
import jax.numpy as jnp
from jax import lax
from jax.experimental import pallas as pl
from jax.experimental.pallas import tpu as pltpu
from jax.experimental.pallas import tpu_sc as plsc


def kernel(x, table):
    B, H = x.shape
    V, D = table.shape
    NBT = B // 128

    t_lin = table
    xt2 = x.T.reshape(H * NBT, 128)

    info = plsc.get_sparse_core_info()
    NC = info.num_cores
    NW = NC * info.num_subcores
    n_pairs = H * NBT
    per_w = n_pairs // NW
    NBUF = 4

    mesh = plsc.VectorSubcoreMesh(core_axis_name="c", subcore_axis_name="s")

    @functools.partial(
        pl.kernel,
        mesh=mesh,
        out_type=jax.ShapeDtypeStruct((H, 8, NBT, 8, 128), jnp.float32),
        compiler_params=pltpu.CompilerParams(
            use_tc_tiling_on_sc=False, needs_layout_passes=False
        ),
        scratch_types=[
            pltpu.VMEM((per_w, 128), jnp.int32),
            pltpu.VMEM((NBUF * 128, D), jnp.float32),
            pltpu.VMEM((2, 8, 8, 128), jnp.float32),
        ] + [pltpu.SemaphoreType.DMA] * (NBUF + 2),
    )
    def emb_kernel(xt_hbm, table_hbm, out_hbm, idx_v, rows_v, trans_v, *sems):
        sem_g = sems[:NBUF]
        sem_w = sems[NBUF:]
        wid = lax.axis_index("s") * NC + lax.axis_index("c")
        iota = lax.iota(jnp.int32, 16)
        base = wid * per_w

        pltpu.sync_copy(xt_hbm.at[pl.ds(base, per_w)], idx_v)

        def fire_gather(p, g):
            pltpu.async_copy(
                table_hbm.at[idx_v.at[p]],
                rows_v.at[pl.ds(g * 128, 128)],
                sem_g[g],
            )

        def wait_gather(g):
            pltpu.make_async_copy(
                table_hbm.at[idx_v.at[0]],
                rows_v.at[pl.ds(g * 128, 128)],
                sem_g[g],
            ).wait()

        def transpose(g, t):
            def fg_body(fg, carry):
                for f in range(8):
                    c = fg * 8 + f
                    col = lax.rem(iota + c, 64)
                    fgv = lax.shift_right_logical(col, 3)
                    fv = lax.bitwise_and(col, 7)
                    for tg in range(8):
                        vec = plsc.load_gather(
                            rows_v, [g * 128 + tg * 16 + iota, col]
                        )
                        plsc.store_scatter(
                            trans_v.at[t], [fgv, fv, tg * 16 + iota], vec
                        )
                return carry

            lax.fori_loop(0, 8, fg_body, None)

        def start_write(p, t):
            pltpu.async_copy(
                trans_v.at[t],
                out_hbm.at[(base + p) // NBT, :, (base + p) % NBT],
                sem_w[t],
            )

        def wait_write(t):
            pltpu.make_async_copy(
                trans_v.at[t], out_hbm.at[0, :, 0], sem_w[t]
            ).wait()

        for g in range(NBUF):
            fire_gather(g, g)

        def body(q, carry):
            for r in range(NBUF):
                p = q * NBUF + r
                g = r
                t = r % 2
                wait_gather(g)

                @pl.when(p >= 2)
                def _():
                    wait_write(t)

                transpose(g, t)

                @pl.when(p + NBUF < per_w)
                def _():
                    fire_gather(p + NBUF, g)

                start_write(p, t)
            return carry

        lax.fori_loop(0, per_w // NBUF, body, None)
        wait_write(0)
        wait_write(1)

    out5 = emb_kernel(xt2, t_lin)
    return out5.transpose(2, 4, 0, 1, 3).reshape(B, H, D)
